# Initial kernel scaffold; baseline (speedup 1.0000x reference)
import jax
import jax.numpy as jnp
from jax.experimental import pallas as pl


def kernel(trg_token_ids_batch):
    raise NotImplementedError("write your pallas kernel here")


# speedup vs baseline: 8.5084x; 8.5084x over previous
import jax
import jax.numpy as jnp
from jax.experimental import pallas as pl

_SMOOTHING = 0.1
_CONFIDENCE = 1.0 - _SMOOTHING
_PAD = 0
_VOCAB = 32000
_SMOOTH_VAL = _SMOOTHING / (_VOCAB - 2)

_ROW_BLK = 256
_COL_BLK = 6400


def _smooth_kernel(ids_ref, out_ref):
    j = pl.program_id(1)
    ids = ids_ref[:, 0]
    tgt = ids[:, None]
    col = jax.lax.broadcasted_iota(jnp.int32, (_ROW_BLK, _COL_BLK), 1) + j * _COL_BLK
    val = jnp.where(col == tgt, _CONFIDENCE, _SMOOTH_VAL)
    zero = (col == _PAD) | (tgt == _PAD)
    out_ref[...] = jnp.where(zero, 0.0, val).astype(jnp.float32)


def kernel(trg_token_ids_batch):
    batch = trg_token_ids_batch.shape[0]
    grid = (batch // _ROW_BLK, _VOCAB // _COL_BLK)
    return pl.pallas_call(
        _smooth_kernel,
        grid=grid,
        in_specs=[pl.BlockSpec((_ROW_BLK, 1), lambda i, j: (i, 0))],
        out_specs=pl.BlockSpec((_ROW_BLK, _COL_BLK), lambda i, j: (i, j)),
        out_shape=jax.ShapeDtypeStruct((batch, _VOCAB), jnp.float32),
    )(trg_token_ids_batch)
